# Initial kernel scaffold; baseline (speedup 1.0000x reference)
import jax
import jax.numpy as jnp
from jax.experimental import pallas as pl


def kernel(features, weight, nodes, neigh_idx):
    raise NotImplementedError("write your pallas kernel here")


# speedup vs baseline: 1.9591x; 1.9591x over previous
import functools

import jax
import jax.numpy as jnp
from jax import lax
from jax.experimental import pallas as pl
from jax.experimental.pallas import tpu as pltpu
from jax.experimental.pallas import tpu_sc as plsc

NC = 2
NS = 16
NW = NC * NS

FEAT = 256
K = 10
CHS = 64
CHN = 8


def _sc_gather_fn(b_pad):
    b_per_w = b_pad // NW
    ns_chunks = b_per_w // CHS
    nn_chunks = b_per_w // CHN
    mesh = plsc.VectorSubcoreMesh(core_axis_name="c", subcore_axis_name="s")

    @functools.partial(
        pl.kernel,
        mesh=mesh,
        out_type=(
            jax.ShapeDtypeStruct((b_pad, FEAT), jnp.float32),
            jax.ShapeDtypeStruct((b_pad, FEAT), jnp.float32),
        ),
        scratch_types=(
            pltpu.VMEM((b_per_w,), jnp.int32),
            pltpu.VMEM((b_per_w * K,), jnp.int32),
            pltpu.VMEM((CHS, FEAT), jnp.float32),
            pltpu.VMEM((CHN * K, FEAT), jnp.float32),
            pltpu.VMEM((CHN * K, FEAT), jnp.float32),
            pltpu.VMEM((CHN, FEAT), jnp.float32),
            pltpu.VMEM((CHN, FEAT), jnp.float32),
            pltpu.SemaphoreType.DMA,
            pltpu.SemaphoreType.DMA,
            pltpu.SemaphoreType.DMA,
        ),
    )
    def sc_gather(feat_hbm, nodes_hbm, neigh_hbm, self_out, neigh_out,
                  idx_s, idx_n, rows_s, nbuf0, nbuf1, stage0, stage1,
                  sem_s, sem_n0, sem_n1):
        wid = lax.axis_index("s") * NC + lax.axis_index("c")
        base = wid * b_per_w
        pltpu.sync_copy(nodes_hbm.at[pl.ds(base, b_per_w)], idx_s)
        pltpu.sync_copy(neigh_hbm.at[pl.ds(base * K, b_per_w * K)], idx_n)

        def n_src(i):
            return feat_hbm.at[idx_n.at[pl.ds(i * (CHN * K), CHN * K)]]

        def reduce_chunk(buf, stage):
            @pl.loop(0, CHN)
            def _col(c):
                rbase = c * K
                for d in range(FEAT // 16):
                    sl = pl.ds(d * 16, 16)
                    acc = buf[rbase, sl]
                    for j in range(1, K):
                        acc = acc + buf[rbase + j, sl]
                    stage[c, sl] = acc * jnp.float32(1.0 / K)

        pltpu.async_copy(n_src(0), nbuf0, sem_n0)
        pltpu.async_copy(n_src(1), nbuf1, sem_n1)

        @pl.loop(0, ns_chunks)
        def _self(i):
            off = i * CHS
            pltpu.async_copy(
                feat_hbm.at[idx_s.at[pl.ds(off, CHS)]], rows_s, sem_s).wait()
            pltpu.sync_copy(rows_s, self_out.at[pl.ds(base + off, CHS)])

        @pl.loop(0, nn_chunks, step=2)
        def _neigh(i):
            for b, (buf, stage, sem) in enumerate(
                    ((nbuf0, stage0, sem_n0), (nbuf1, stage1, sem_n1))):
                ic = i + b
                pltpu.make_async_copy(n_src(ic), buf, sem).wait()
                reduce_chunk(buf, stage)

                @pl.when(ic + 2 < nn_chunks)
                def _refire():
                    pltpu.async_copy(n_src(ic + 2), buf, sem)

                pltpu.sync_copy(
                    stage, neigh_out.at[pl.ds(base + ic * CHN, CHN)])

    return sc_gather


def _tc_body(w_ref, s_ref, n_ref, o_ref):
    w = w_ref[...]
    s = s_ref[...]
    n = n_ref[...]
    dn = (((1,), (1,)), ((), ()))
    acc = lax.dot_general(w[:, :FEAT], s, dn, preferred_element_type=jnp.float32)
    acc = acc + lax.dot_general(w[:, FEAT:], n, dn,
                                preferred_element_type=jnp.float32)
    o_ref[...] = jnp.maximum(acc, 0.0)


def _tc_matmul(weight, self_f, neigh_m, b_pad, tb):
    grid = (b_pad // tb,)
    return pl.pallas_call(
        _tc_body,
        grid=grid,
        in_specs=[
            pl.BlockSpec((FEAT, 2 * FEAT), lambda i: (0, 0)),
            pl.BlockSpec((tb, FEAT), lambda i: (i, 0)),
            pl.BlockSpec((tb, FEAT), lambda i: (i, 0)),
        ],
        out_specs=pl.BlockSpec((FEAT, tb), lambda i: (0, i)),
        out_shape=jax.ShapeDtypeStruct((FEAT, b_pad), jnp.float32),
    )(weight, self_f, neigh_m)


def kernel(features, weight, nodes, neigh_idx):
    b = nodes.shape[0]
    b_pad = ((b + NW * CHS - 1) // (NW * CHS)) * (NW * CHS)

    nodes_p = jnp.zeros((b_pad,), jnp.int32).at[:b].set(nodes.astype(jnp.int32))
    neigh_p = jnp.zeros((b_pad, K), jnp.int32).at[:b].set(
        neigh_idx.astype(jnp.int32))
    neigh_flat = neigh_p.reshape(b_pad * K)

    self_f, neigh_m = _sc_gather_fn(b_pad)(features, nodes_p, neigh_flat)
    out = _tc_matmul(weight, self_f, neigh_m, b_pad, tb=1024)
    return out[:, :b]


# speedup vs baseline: 1.9891x; 1.0153x over previous
import functools

import jax
import jax.numpy as jnp
from jax import lax
from jax.experimental import pallas as pl
from jax.experimental.pallas import tpu as pltpu
from jax.experimental.pallas import tpu_sc as plsc

NC = 2
NS = 16
NW = NC * NS

FEAT = 256
NLANE = 16
K = 10
G = K + 1
CHN = 16
NSTREAM = 2
CPS = CHN // NSTREAM


def _sc_gather_fn(b_pad):
    b_per_w = b_pad // NW
    n_chunks = b_per_w // CHN
    mesh = plsc.VectorSubcoreMesh(core_axis_name="c", subcore_axis_name="s")

    @functools.partial(
        pl.kernel,
        mesh=mesh,
        out_type=(
            jax.ShapeDtypeStruct((b_pad, FEAT), jnp.float32),
            jax.ShapeDtypeStruct((b_pad, FEAT), jnp.float32),
        ),
        scratch_types=(
            pltpu.VMEM((b_per_w * G,), jnp.int32),
            pltpu.VMEM((CHN * G, FEAT), jnp.float32),
            pltpu.VMEM((CHN * G, FEAT), jnp.float32),
            pltpu.VMEM((CHN, FEAT), jnp.float32),
            pltpu.VMEM((CHN, FEAT), jnp.float32),
            pltpu.VMEM((CHN, FEAT), jnp.float32),
            pltpu.VMEM((CHN, FEAT), jnp.float32),
            pltpu.SemaphoreType.DMA,
            pltpu.SemaphoreType.DMA,
            pltpu.SemaphoreType.DMA,
            pltpu.SemaphoreType.DMA,
        ),
    )
    def sc_gather(feat_hbm, idx_hbm, self_out, neigh_out,
                  idx_v, buf0, buf1, ss0, ss1, ns0, ns1,
                  sem_i0, sem_i1, sem_o0, sem_o1):
        wid = lax.axis_index("s") * NC + lax.axis_index("c")
        base = wid * b_per_w
        pltpu.sync_copy(idx_hbm.at[pl.ds(base * G, b_per_w * G)], idx_v)

        def in_copies(ic, buf, sem):
            return [
                pltpu.make_async_copy(
                    feat_hbm.at[idx_v.at[pl.ds((ic * CHN + s * CPS) * G,
                                               CPS * G)]],
                    buf.at[pl.ds(s * CPS * G, CPS * G)],
                    sem)
                for s in range(NSTREAM)
            ]

        def out_copies(ic, sstage, nstage, sem):
            dst = pl.ds(base + ic * CHN, CHN)
            return [
                pltpu.make_async_copy(sstage, self_out.at[dst], sem),
                pltpu.make_async_copy(nstage, neigh_out.at[dst], sem),
            ]

        def reduce_chunk(buf, sstage, nstage):
            @pl.loop(0, CHN)
            def _col(c):
                rbase = c * G
                for d in range(FEAT // NLANE):
                    sl = pl.ds(d * NLANE, NLANE)
                    sstage[c, sl] = buf[rbase, sl]
                    acc = buf[rbase + 1, sl]
                    for j in range(2, G):
                        acc = acc + buf[rbase + j, sl]
                    nstage[c, sl] = acc * jnp.float32(1.0 / K)

        slots = ((buf0, ss0, ns0, sem_i0, sem_o0),
                 (buf1, ss1, ns1, sem_i1, sem_o1))

        for b, (buf, _, _, sem_i, _) in enumerate(slots):
            for c in in_copies(b, buf, sem_i):
                c.start()

        @pl.loop(0, n_chunks, step=2)
        def _chunk(i):
            for b, (buf, sstage, nstage, sem_i, sem_o) in enumerate(slots):
                ic = i + b
                for c in in_copies(ic, buf, sem_i):
                    c.wait()

                @pl.when(ic >= 2)
                def _drain():
                    for c in out_copies(ic - 2, sstage, nstage, sem_o):
                        c.wait()

                reduce_chunk(buf, sstage, nstage)

                @pl.when(ic + 2 < n_chunks)
                def _refire():
                    for c in in_copies(ic + 2, buf, sem_i):
                        c.start()

                for c in out_copies(ic, sstage, nstage, sem_o):
                    c.start()

        for b, (buf, sstage, nstage, _, sem_o) in enumerate(slots):
            for c in out_copies(n_chunks - 2 + b, sstage, nstage, sem_o):
                c.wait()

    return sc_gather


def _tc_body(w_ref, s_ref, n_ref, o_ref):
    w = w_ref[...]
    s = s_ref[...]
    n = n_ref[...]
    dn = (((1,), (1,)), ((), ()))
    acc = lax.dot_general(w[:, :FEAT], s, dn, preferred_element_type=jnp.float32)
    acc = acc + lax.dot_general(w[:, FEAT:], n, dn,
                                preferred_element_type=jnp.float32)
    o_ref[...] = jnp.maximum(acc, 0.0)


def _tc_matmul(weight, self_f, neigh_m, b_pad, tb):
    grid = (b_pad // tb,)
    return pl.pallas_call(
        _tc_body,
        grid=grid,
        in_specs=[
            pl.BlockSpec((FEAT, 2 * FEAT), lambda i: (0, 0)),
            pl.BlockSpec((tb, FEAT), lambda i: (i, 0)),
            pl.BlockSpec((tb, FEAT), lambda i: (i, 0)),
        ],
        out_specs=pl.BlockSpec((FEAT, tb), lambda i: (0, i)),
        out_shape=jax.ShapeDtypeStruct((FEAT, b_pad), jnp.float32),
    )(weight, self_f, neigh_m)


def kernel(features, weight, nodes, neigh_idx):
    b = nodes.shape[0]
    align = NW * CHN * 4
    b_pad = ((b + align - 1) // align) * align

    nodes_p = jnp.zeros((b_pad,), jnp.int32).at[:b].set(nodes.astype(jnp.int32))
    neigh_p = jnp.zeros((b_pad, K), jnp.int32).at[:b].set(
        neigh_idx.astype(jnp.int32))
    idx_flat = jnp.concatenate([nodes_p[:, None], neigh_p], axis=1).reshape(-1)

    self_f, neigh_m = _sc_gather_fn(b_pad)(features, idx_flat)
    out = _tc_matmul(weight, self_f, neigh_m, b_pad, tb=1024)
    return out[:, :b]
